# baseline (speedup 1.0000x reference)
import functools

import jax
import jax.numpy as jnp
from jax import lax
from jax.experimental import pallas as pl
from jax.experimental.pallas import tpu as pltpu
from jax.experimental.pallas import tpu_sc as plsc

N = 16384
K = 1000
B = 200
L = 16
CHUNK_OFFS = tuple(range(0, B - L, L)) + (B - L,)
NC = 2
NS = 16
NW = NC * NS
ROWS_PER_W = N // NW
R = 32
NB = ROWS_PER_W // R

_mesh = plsc.VectorSubcoreMesh(core_axis_name="c", subcore_axis_name="s")


@functools.partial(
    pl.kernel,
    mesh=_mesh,
    compiler_params=pltpu.CompilerParams(
        needs_layout_passes=False,
        use_tc_tiling_on_sc=True,
    ),
    out_type=jax.ShapeDtypeStruct((N, B), jnp.float32),
    scratch_types=[
        pltpu.VMEM((R, K), jnp.float32),
        pltpu.VMEM((R, K), jnp.float32),
        pltpu.VMEM((R, B), jnp.int32),
        pltpu.VMEM((R, B), jnp.int32),
        pltpu.VMEM((R, B), jnp.float32),
        pltpu.VMEM((R, B), jnp.float32),
        pltpu.SemaphoreType.DMA,
        pltpu.SemaphoreType.DMA,
        pltpu.SemaphoreType.DMA,
        pltpu.SemaphoreType.DMA,
    ],
)
def _gather_rows(x_hbm, y_hbm, o_hbm,
                 x0, x1, y0, y1, o0, o1, si0, si1, so0, so1):
    wid = lax.axis_index("s") * NC + lax.axis_index("c")
    base0 = wid * ROWS_PER_W
    xs, ys, os_ = (x0, x1), (y0, y1), (o0, o1)
    sis, sos = (si0, si1), (so0, so1)

    def in_copies(g, b):
        base = base0 + g * R
        cx = pltpu.make_async_copy(
            x_hbm.at[pl.ds(base, R), :], xs[b], sis[b])
        cy = pltpu.make_async_copy(
            y_hbm.at[pl.ds(base, R), :], ys[b], sis[b])
        return cx, cy

    def out_copy(g, b):
        base = base0 + g * R
        return pltpu.make_async_copy(
            os_[b], o_hbm.at[pl.ds(base, R), :], sos[b])

    cx, cy = in_copies(0, 0)
    cx.start()
    cy.start()

    def pair(it, carry):
        for par in range(2):
            g = it * 2 + par

            @pl.when(g + 1 < NB)
            def _():
                nx, ny = in_copies(g + 1, 1 - par)
                nx.start()
                ny.start()

            cx, cy = in_copies(g, par)
            cx.wait()
            cy.wait()

            @pl.when(g >= 2)
            def _():
                out_copy(g - 2, par).wait()

            x_v, y_v, o_v = xs[par], ys[par], os_[par]

            @plsc.parallel_loop(0, R, 1, unroll=4)
            def row(i):
                rvec = jnp.full((L,), i, dtype=jnp.int32)
                for off in CHUNK_OFFS:
                    idx = y_v[i, pl.ds(off, L)]
                    vals = plsc.load_gather(x_v, [rvec, idx])
                    o_v[i, pl.ds(off, L)] = vals

            out_copy(g, par).start()
        return carry

    lax.fori_loop(0, NB // 2, pair, 0)
    out_copy(NB - 2, 0).wait()
    out_copy(NB - 1, 1).wait()


def kernel(x, y):
    return _gather_rows(x, y.astype(jnp.int32))


# speedup vs baseline: 1.2609x; 1.2609x over previous
import functools

import jax
import jax.numpy as jnp
from jax import lax
from jax.experimental import pallas as pl
from jax.experimental.pallas import tpu as pltpu
from jax.experimental.pallas import tpu_sc as plsc

N = 16384
K = 1000
B = 200
L = 16
NC = 2
NS = 16
NW = NC * NS
ROWS_PER_W = N // NW
R = 32
RS = 128
SUB = RS // R
NSB = ROWS_PER_W // RS
NXB = ROWS_PER_W // R

_mesh = plsc.VectorSubcoreMesh(core_axis_name="c", subcore_axis_name="s")


@functools.partial(
    pl.kernel,
    mesh=_mesh,
    compiler_params=pltpu.CompilerParams(
        needs_layout_passes=False,
        use_tc_tiling_on_sc=True,
    ),
    out_type=jax.ShapeDtypeStruct((B, N), jnp.float32),
    scratch_types=[
        pltpu.VMEM((R, K), jnp.float32),
        pltpu.VMEM((R, K), jnp.float32),
        pltpu.VMEM((B, RS), jnp.int32),
        pltpu.VMEM((B, RS), jnp.float32),
        pltpu.SemaphoreType.DMA,
        pltpu.SemaphoreType.DMA,
        pltpu.SemaphoreType.DMA,
        pltpu.SemaphoreType.DMA,
    ],
)
def _gather_rows(x_hbm, yT_hbm, oT_hbm, x0, x1, y_v, o_v, sx0, sx1, sy, so):
    wid = lax.axis_index("s") * NC + lax.axis_index("c")
    r0 = wid * ROWS_PER_W
    xs, sxs = (x0, x1), (sx0, sx1)
    iota = lax.iota(jnp.int32, L)

    def x_copy(gk, b):
        return pltpu.make_async_copy(
            x_hbm.at[pl.ds(r0 + gk * R, R), :], xs[b], sxs[b])

    def y_copy(s):
        return pltpu.make_async_copy(
            yT_hbm.at[:, pl.ds(r0 + s * RS, RS)], y_v, sy)

    def o_copy(s):
        return pltpu.make_async_copy(
            o_v, oT_hbm.at[:, pl.ds(r0 + s * RS, RS)], so)

    y_copy(0).start()
    x_copy(0, 0).start()

    for s in range(NSB):
        y_copy(s).wait()
        if s > 0:
            o_copy(s - 1).wait()
        for k in range(SUB):
            gk = s * SUB + k
            if gk + 1 < NXB:
                x_copy(gk + 1, (gk + 1) % 2).start()
            x_copy(gk, gk % 2).wait()
            x_v = xs[gk % 2]

            @plsc.parallel_loop(0, B, 1, unroll=2)
            def jrow(j):
                for g in range(R // L):
                    col = k * R + g * L
                    idx = y_v[j, pl.ds(col, L)]
                    vals = plsc.load_gather(x_v, [iota + (g * L), idx])
                    o_v[j, pl.ds(col, L)] = vals

        if s + 1 < NSB:
            y_copy(s + 1).start()
        o_copy(s).start()

    o_copy(NSB - 1).wait()


def kernel(x, y):
    yT = jnp.swapaxes(y.astype(jnp.int32), 0, 1)
    outT = _gather_rows(x, yT)
    return jnp.swapaxes(outT, 0, 1)
